# baseline (speedup 1.0000x reference)
import functools

import jax
import jax.numpy as jnp
from jax import lax
from jax.experimental import pallas as pl
from jax.experimental.pallas import tpu as pltpu
from jax.experimental.pallas import tpu_sc as plsc

N = 10000
NP = 10240
D = 256
DH = 128
NC = 2
NS = 16
CH = 128
CPB = 8
ROWS_PER_SUB = NP // NS
ZROWS = 128

_mesh = plsc.VectorSubcoreMesh(
    core_axis_name="c", subcore_axis_name="s", num_cores=NC, num_subcores=NS)


@functools.lru_cache(maxsize=None)
def _make_spmm(nblocks):
    @functools.partial(
        pl.kernel,
        out_type=(jax.ShapeDtypeStruct((NP, DH), jnp.float32),
                  jax.ShapeDtypeStruct((NP, DH), jnp.float32)),
        mesh=_mesh,
        scratch_types=[
            pltpu.VMEM((2, CPB, CH), jnp.int32),
            pltpu.VMEM((CPB * CH,), jnp.float32),
            pltpu.VMEM((CH, DH), jnp.float32),
            pltpu.VMEM((CH, DH), jnp.float32),
            pltpu.VMEM_SHARED((NP, DH), jnp.float32),
            pltpu.SemaphoreType.DMA,
            pltpu.SemaphoreType.DMA,
        ],
    )
    def spmm(src_lo, src_hi, edata, wdata, out_lo, out_hi,
             ebuf, wbuf, gbuf0, gbuf1, acc, gsem0, gsem1):
        c = lax.axis_index("c")
        s = lax.axis_index("s")

        def zrow(r, carry):
            for jj in range(DH // 16):
                gbuf0[r, pl.ds(jj * 16, 16)] = jnp.zeros((16,), jnp.float32)
            return carry
        lax.fori_loop(0, ZROWS, zrow, 0)
        for k in range(ROWS_PER_SUB // ZROWS):
            pltpu.sync_copy(
                gbuf0, acc.at[pl.ds(s * ROWS_PER_SUB + k * ZROWS, ZROWS)])
        plsc.subcore_barrier()

        def run(src_hbm, out_hbm):
            def scale(gref, k):
                def grp(g, carry):
                    woff = pl.multiple_of(k * CH + g * 16, 16)
                    wv16 = wbuf[pl.ds(woff, 16)]
                    for l in range(16):
                        wb = lax.gather(
                            wv16, jnp.full((16, 1), l, jnp.int32),
                            lax.GatherDimensionNumbers(
                                offset_dims=(), collapsed_slice_dims=(0,),
                                start_index_map=(0,)),
                            (1,),
                            mode=lax.GatherScatterMode.PROMISE_IN_BOUNDS)
                        i = g * 16 + l
                        for jj in range(DH // 16):
                            sl = pl.ds(jj * 16, 16)
                            gref[i, sl] = gref[i, sl] * wb
                    return carry
                lax.fori_loop(0, CH // 16, grp, 0)

            def block(bi, carry):
                pltpu.sync_copy(edata.at[s, bi], ebuf)
                pltpu.sync_copy(wdata.at[s, bi], wbuf)
                pltpu.async_copy(src_hbm.at[ebuf.at[1, 0]], gbuf0, gsem0)

                def pair(p, pcarry):
                    k0 = p * 2
                    pltpu.async_copy(
                        src_hbm.at[ebuf.at[1, k0 + 1]], gbuf1, gsem1)
                    pltpu.make_async_copy(
                        src_hbm.at[ebuf.at[1, k0]], gbuf0, gsem0).wait()
                    scale(gbuf0, k0)
                    pltpu.sync_copy(gbuf0, acc.at[ebuf.at[0, k0]], add=True)

                    @pl.when(p < CPB // 2 - 1)
                    def _():
                        pltpu.async_copy(
                            src_hbm.at[ebuf.at[1, k0 + 2]], gbuf0, gsem0)
                    pltpu.make_async_copy(
                        src_hbm.at[ebuf.at[1, k0 + 1]], gbuf1, gsem1).wait()
                    scale(gbuf1, k0 + 1)
                    pltpu.sync_copy(gbuf1, acc.at[ebuf.at[0, k0 + 1]], add=True)
                    return pcarry
                lax.fori_loop(0, CPB // 2, pair, 0)
                return carry
            lax.fori_loop(0, nblocks, block, 0)
            plsc.subcore_barrier()
            base = s * ROWS_PER_SUB
            pltpu.sync_copy(acc.at[pl.ds(base, ROWS_PER_SUB)],
                            out_hbm.at[pl.ds(base, ROWS_PER_SUB)])

        @pl.when(c == 0)
        def _():
            run(src_lo, out_lo)

        @pl.when(c == 1)
        def _():
            run(src_hi, out_hi)

    return spmm


BN = 400


def _dense_body(x_b, h1lo_b, h1hi_b, h2lo_b, h2hi_b,
                wx, w1lo, w1hi, w2lo, w2hi, b_b, out_b):
    acc = jnp.dot(x_b[...], wx[...], preferred_element_type=jnp.float32)
    acc += jnp.dot(h1lo_b[...], w1lo[...], preferred_element_type=jnp.float32)
    acc += jnp.dot(h1hi_b[...], w1hi[...], preferred_element_type=jnp.float32)
    acc += jnp.dot(h2lo_b[...], w2lo[...], preferred_element_type=jnp.float32)
    acc += jnp.dot(h2hi_b[...], w2hi[...], preferred_element_type=jnp.float32)
    out_b[...] = acc + b_b[...]


_dense = pl.pallas_call(
    _dense_body,
    grid=(N // BN,),
    in_specs=[
        pl.BlockSpec((BN, D), lambda i: (i, 0)),
        pl.BlockSpec((BN, DH), lambda i: (i, 0)),
        pl.BlockSpec((BN, DH), lambda i: (i, 0)),
        pl.BlockSpec((BN, DH), lambda i: (i, 0)),
        pl.BlockSpec((BN, DH), lambda i: (i, 0)),
        pl.BlockSpec((D, D), lambda i: (0, 0)),
        pl.BlockSpec((DH, D), lambda i: (0, 0)),
        pl.BlockSpec((DH, D), lambda i: (0, 0)),
        pl.BlockSpec((DH, D), lambda i: (0, 0)),
        pl.BlockSpec((DH, D), lambda i: (0, 0)),
        pl.BlockSpec((1, D), lambda i: (0, 0)),
    ],
    out_specs=pl.BlockSpec((BN, D), lambda i: (i, 0)),
    out_shape=jax.ShapeDtypeStruct((N, D), jnp.float32),
)


def kernel(x, edge_index, edge_weight, W, b):
    e = edge_index.shape[1]
    eb = NS * CPB * CH
    nblocks = -(-e // eb)
    ep = eb * nblocks
    rows = jnp.pad(edge_index[0], (0, ep - e))
    cols = jnp.pad(edge_index[1], (0, ep - e))
    w = jnp.pad(edge_weight, (0, ep - e))
    edata = jnp.stack([
        rows.reshape(NS, nblocks, CPB, CH),
        cols.reshape(NS, nblocks, CPB, CH),
    ], axis=2)
    wdata = w.reshape(NS, nblocks, CPB * CH)

    x_lo = x[:, :DH]
    x_hi = x[:, DH:]
    spmm = _make_spmm(nblocks)
    h1_lo, h1_hi = spmm(x_lo, x_hi, edata, wdata)
    h2_lo, h2_hi = spmm(h1_lo, h1_hi, edata, wdata)

    wt = W.T
    out = _dense(x, h1_lo[:N], h1_hi[:N], h2_lo[:N], h2_hi[:N],
                 wt[:D], wt[D:D + DH], wt[D + DH:2 * D],
                 wt[2 * D:2 * D + DH], wt[2 * D + DH:],
                 b.reshape(1, D))
    return out
